# baseline (speedup 1.0000x reference)
import numpy as np
import jax
import jax.numpy as jnp
from jax.experimental import pallas as pl
from jax.experimental.pallas import tpu as pltpu


def _tf2x32(k1, k2, x0, x1):
    ks0, ks1 = k1, k2
    ks2 = k1 ^ k2 ^ np.uint32(0x1BD11BDA)
    rot1 = (13, 15, 26, 6)
    rot2 = (17, 29, 16, 24)

    def rnd(a, b, r):
        a = a + b
        b = (b << np.uint32(r)) | (b >> np.uint32(32 - r))
        b = a ^ b
        return a, b

    x0 = x0 + ks0
    x1 = x1 + ks1
    for r in rot1:
        x0, x1 = rnd(x0, x1, r)
    x0 = x0 + ks1; x1 = x1 + ks2 + np.uint32(1)
    for r in rot2:
        x0, x1 = rnd(x0, x1, r)
    x0 = x0 + ks2; x1 = x1 + ks0 + np.uint32(2)
    for r in rot1:
        x0, x1 = rnd(x0, x1, r)
    x0 = x0 + ks0; x1 = x1 + ks1 + np.uint32(3)
    for r in rot2:
        x0, x1 = rnd(x0, x1, r)
    x0 = x0 + ks1; x1 = x1 + ks2 + np.uint32(4)
    for r in rot1:
        x0, x1 = rnd(x0, x1, r)
    x0 = x0 + ks2; x1 = x1 + ks0 + np.uint32(5)
    return x0, x1


def _np_key_constants():
    k1 = np.uint32(0)
    k2 = np.uint32(42)
    chi = np.array([0, 0], dtype=np.uint32)
    clo = np.array([0, 1], dtype=np.uint32)
    b1, b2 = _tf2x32(k1, k2, chi, clo)
    return (b1[0], b2[0]), (b1[1], b2[1])


_K_NOISE, _K_RES = _np_key_constants()

_TINY = np.float32(1.1754943508222875e-38)
_INF = np.float32(np.inf)


def _stageb_body(x_ref, n_ref, tt_ref, ct_ref, ft_ref, obs_ref, npart_ref,
                 upd_ref, v_ref):
    b = pl.program_id(0)
    x = x_ref[...]
    n = n_ref[...]
    upd = jnp.dot(x, tt_ref[...], preferred_element_type=jnp.float32)
    upd = upd + jnp.dot(n, ct_ref[...], preferred_element_type=jnp.float32)
    pred = jnp.dot(upd, ft_ref[...], preferred_element_type=jnp.float32)
    d = obs_ref[...] - pred
    w = jnp.sum(d * d, axis=1, keepdims=True)
    bc = x.shape[0]
    g = b * bc + jax.lax.broadcasted_iota(jnp.int32, (bc, 1), 0)
    v = jnp.where(g < npart_ref[0], jnp.float32(1.0) / (w + np.float32(1e-30)),
                  _INF)
    upd_ref[...] = upd
    v_ref[...] = v


def _make_stagec(n_particles, cp, br, cw):
    nch = cp // cw
    gr = n_particles // br
    ncols = np.uint32(n_particles)

    def body(lo0_ref, hi0_ref, v_ref, out_ref):
        lo0 = lo0_ref[0]
        hi0 = hi0_ref[0]
        k1 = _K_RES[0]
        k2 = _K_RES[1]

        def chunk(c, carry):
            smin, sidx = carry
            j0 = c * cw
            ji = j0 + jax.lax.broadcasted_iota(jnp.int32, (1, cw), 1)
            ju = ji.astype(jnp.uint32)
            lo = lo0 + ju
            cr = (lo < ju).astype(jnp.uint32)
            hi = hi0 + cr
            b1, b2 = _tf2x32(k1, k2, hi, lo)
            bits = b1 ^ b2
            f = jax.lax.bitcast_convert_type(
                (bits >> np.uint32(9)) | np.uint32(0x3F800000), jnp.float32)
            u = jnp.maximum(f - np.float32(1.0), _TINY)
            e = -jnp.log(u)
            s = e * v_ref[c]
            m = s < smin
            smin = jnp.where(m, s, smin)
            sidx = jnp.where(m, jnp.broadcast_to(ji, (br, cw)), sidx)
            return smin, sidx

        smin = jnp.full((br, cw), _INF, dtype=jnp.float32)
        sidx = jnp.zeros((br, cw), dtype=jnp.int32)
        smin, sidx = jax.lax.fori_loop(0, nch, chunk, (smin, sidx))
        rowmin = jnp.min(smin, axis=1, keepdims=True)
        idx = jnp.min(jnp.where(smin == rowmin, sidx, jnp.int32(0x7FFFFFFF)),
                      axis=1)
        out_ref[0, 0, :] = idx

    call = pl.pallas_call(
        body,
        grid=(gr,),
        in_specs=[
            pl.BlockSpec((1, br, 1), lambda b: (b, 0, 0)),
            pl.BlockSpec((1, br, 1), lambda b: (b, 0, 0)),
            pl.BlockSpec((nch, 1, cw), lambda b: (0, 0, 0)),
        ],
        out_specs=pl.BlockSpec((1, 1, br), lambda b: (b, 0, 0)),
        out_shape=jax.ShapeDtypeStruct((gr, 1, br), jnp.int32),
    )
    return call, gr, nch


def kernel(inputs, state_vector, transition_matrix, process_noise_cov,
           forward_matrix):
    n = state_vector.shape[0]
    cw = 1024
    br = 8
    cp = ((n + cw - 1) // cw) * cw
    bc = min(cp, 2048)
    while cp % bc:
        bc //= 2

    key = jax.random.key(42)
    k_noise, _ = jax.random.split(key)
    noise_raw = jax.random.normal(k_noise, state_vector.shape,
                                  dtype=state_vector.dtype)
    chol = jnp.linalg.cholesky(process_noise_cov)

    pad = cp - n
    xp = jnp.pad(state_vector, ((0, pad), (0, 0)))
    npd = jnp.pad(noise_raw, ((0, pad), (0, 0)))

    upd, v = pl.pallas_call(
        _stageb_body,
        grid=(cp // bc,),
        in_specs=[
            pl.BlockSpec((bc, 3), lambda b: (b, 0)),
            pl.BlockSpec((bc, 3), lambda b: (b, 0)),
            pl.BlockSpec((3, 3), lambda b: (0, 0)),
            pl.BlockSpec((3, 3), lambda b: (0, 0)),
            pl.BlockSpec((3, 64), lambda b: (0, 0)),
            pl.BlockSpec((1, 64), lambda b: (0, 0)),
            pl.BlockSpec(memory_space=pltpu.SMEM),
        ],
        out_specs=[
            pl.BlockSpec((bc, 3), lambda b: (b, 0)),
            pl.BlockSpec((bc, 1), lambda b: (b, 0)),
        ],
        out_shape=[
            jax.ShapeDtypeStruct((cp, 3), jnp.float32),
            jax.ShapeDtypeStruct((cp, 1), jnp.float32),
        ],
    )(xp, npd, transition_matrix.T, chol.T, forward_matrix.T,
      inputs.reshape(1, 64), jnp.array([n], dtype=jnp.int32))

    rows = jnp.arange(n, dtype=jnp.uint32)
    nn = np.uint32(n)
    a = (rows >> np.uint32(12)) * nn
    b_ = (rows & np.uint32(0xFFF)) * nn
    lo0 = (a << np.uint32(12)) + b_
    c0 = (lo0 < b_).astype(jnp.uint32)
    hi0 = (a >> np.uint32(20)) + c0

    callc, gr, nch = _make_stagec(n, cp, br, cw)
    idx3 = callc(lo0.reshape(gr, br, 1), hi0.reshape(gr, br, 1),
                 v.reshape(nch, 1, cw))
    idx = idx3.reshape(n)

    resampled = jnp.take(upd[:n], idx, axis=0)
    return jnp.mean(resampled, axis=0)
